# baseline (speedup 1.0000x reference)
import jax
import jax.numpy as jnp
from jax.experimental import pallas as pl
from jax.experimental.pallas import tpu as pltpu

E = 64
TOPK = 2
D = 1024
DFF = 1024
T = 2048
SOFTCAP = 30.0

A = T * TOPK
NP = 5120


def _shift_cumsum(a):
    n = a.shape[0]
    s = 1
    while s < n:
        a = a + jnp.concatenate([jnp.zeros((s,) + a.shape[1:], a.dtype), a[:-s]], axis=0)
        s *= 2
    return a


def _route_body(x_ref, wg_ref, w_ref, pp_ref, off_ref, nb_ref, cnt_ref):
    x = x_ref[...]
    logits = jnp.dot(x, wg_ref[...], preferred_element_type=jnp.float32)
    capped = SOFTCAP * jnp.tanh(logits / SOFTCAP)
    probs = jax.nn.softmax(capped, axis=-1)
    i1 = jnp.argmax(probs, axis=-1)
    w1 = jnp.max(probs, axis=-1)
    cols = jax.lax.broadcasted_iota(jnp.int32, probs.shape, 1)
    masked = jnp.where(cols == i1[:, None], -jnp.inf, probs)
    i2 = jnp.argmax(masked, axis=-1)
    w2 = jnp.max(masked, axis=-1)
    w_ref[...] = jnp.stack([w1, w2], axis=-1)

    flat_e = jnp.concatenate([i1[:, None], i2[:, None]], axis=0).astype(jnp.int32)
    erange = jax.lax.broadcasted_iota(jnp.int32, (A, E), 1)
    oh = (flat_e == erange).astype(jnp.float32)
    ic = _shift_cumsum(oh)
    rank = jnp.sum(ic * oh, axis=-1) - 1.0
    counts = ic[A - 1, :]

    c16 = jnp.floor((counts + 15.0) / 16.0) * 16.0
    tri_lo = (jax.lax.broadcasted_iota(jnp.int32, (E, E), 0)
              < jax.lax.broadcasted_iota(jnp.int32, (E, E), 1)).astype(jnp.float32)
    g16 = jnp.dot(c16[None, :], tri_lo,
                  preferred_element_type=jnp.float32)[0]
    pp = jnp.sum(oh * g16[None, :], axis=-1) + rank
    pp_ref[...] = pp.astype(jnp.int32).reshape(TOPK, T)
    off_ref[...] = g16[None, :].astype(jnp.int32)
    nb_ref[...] = jnp.floor((c16[None, :] + 63.0) / 64.0).astype(jnp.int32)
    cnt_ref[...] = counts[None, :].astype(jnp.int32)


def _route(x, wg):
    return pl.pallas_call(
        _route_body,
        out_shape=(
            jax.ShapeDtypeStruct((T, TOPK), jnp.float32),
            jax.ShapeDtypeStruct((TOPK, T), jnp.int32),
            jax.ShapeDtypeStruct((1, E), jnp.int32),
            jax.ShapeDtypeStruct((1, E), jnp.int32),
            jax.ShapeDtypeStruct((1, E), jnp.int32),
        ),
    )(x, wg)


def _ffn_body(pp_ref, off_ref, nb_ref, cnt_ref, x_hbm, w1_ref, w3_ref, w2_ref,
              ys_hbm, x_v, ys_v, xb_v, yb_v, rid_s, sem_in, sem_out):
    e = pl.program_id(0)

    @pl.when(e == 0)
    def _():
        pltpu.make_async_copy(x_hbm, x_v, sem_in).start()

        def fill(a, _):
            rid_s[pp_ref[a]] = a
            return 0

        jax.lax.fori_loop(0, A, fill, 0)
        pltpu.make_async_copy(x_hbm, x_v, sem_in).wait()

    row0 = off_ref[e]

    def step(k, _):
        base = row0 + 64 * k
        nv = jnp.minimum(cnt_ref[e] - 64 * k, 64)

        def gather(r, _):
            a = rid_s[base + r]
            tok = jax.lax.bitwise_and(a, T - 1)
            xb_v[pl.ds(r, 1), :] = x_v[pl.ds(tok, 1), :]
            return 0

        jax.lax.fori_loop(0, nv, gather, 0)

        xb = xb_v[...]
        h = jax.nn.gelu(
            jnp.dot(xb, w1_ref[0], preferred_element_type=jnp.float32)
        ) * jnp.dot(xb, w3_ref[0], preferred_element_type=jnp.float32)
        yb_v[...] = jnp.dot(h, w2_ref[0], preferred_element_type=jnp.float32)

        def scatter(r, _):
            a = rid_s[base + r]
            ys_v[pl.ds(a, 1), :] = yb_v[pl.ds(r, 1), :]
            return 0

        jax.lax.fori_loop(0, nv, scatter, 0)
        return 0

    jax.lax.fori_loop(0, nb_ref[e], step, 0)

    @pl.when(e == E - 1)
    def _():
        pltpu.make_async_copy(ys_v, ys_hbm, sem_out).start()
        pltpu.make_async_copy(ys_v, ys_hbm, sem_out).wait()


def _ffn(x, w1, w3, w2, pp_flat, off, nb64, cnt):
    grid_spec = pltpu.PrefetchScalarGridSpec(
        num_scalar_prefetch=4,
        grid=(E,),
        in_specs=[
            pl.BlockSpec(memory_space=pltpu.MemorySpace.HBM),
            pl.BlockSpec((1, D, DFF), lambda e, *_: (e, 0, 0)),
            pl.BlockSpec((1, D, DFF), lambda e, *_: (e, 0, 0)),
            pl.BlockSpec((1, DFF, D), lambda e, *_: (e, 0, 0)),
        ],
        out_specs=pl.BlockSpec(memory_space=pltpu.MemorySpace.HBM),
        scratch_shapes=[
            pltpu.VMEM((T, D), jnp.float32),
            pltpu.VMEM((A, D), jnp.float32),
            pltpu.VMEM((64, D), jnp.float32),
            pltpu.VMEM((64, D), jnp.float32),
            pltpu.SMEM((NP,), jnp.int32),
            pltpu.SemaphoreType.DMA,
            pltpu.SemaphoreType.DMA,
        ],
    )
    return pl.pallas_call(
        _ffn_body,
        grid_spec=grid_spec,
        out_shape=jax.ShapeDtypeStruct((A, D), jnp.float32),
    )(pp_flat, off, nb64, cnt, x, w1, w3, w2)


def kernel(hidden_states, Wg, W1, W3, W2):
    x = hidden_states
    topk_w, pp, off, nb64, cnt = _route(x, Wg)

    ys = _ffn(x, W1, W3, W2, pp.reshape(-1), off[0], nb64[0], cnt[0])
    ys = ys.reshape(TOPK, T, D)

    out = topk_w[:, 0:1] * ys[0] + topk_w[:, 1:2] * ys[1]
    return out


# speedup vs baseline: 1.0020x; 1.0020x over previous
import jax
import jax.numpy as jnp
from jax.experimental import pallas as pl
from jax.experimental.pallas import tpu as pltpu

E = 64
TOPK = 2
D = 1024
DFF = 1024
T = 2048
SOFTCAP = 30.0

A = T * TOPK
NP = 5120


def _shift_cumsum(a):
    n = a.shape[0]
    s = 1
    while s < n:
        a = a + jnp.concatenate([jnp.zeros((s,) + a.shape[1:], a.dtype), a[:-s]], axis=0)
        s *= 2
    return a


def _route_body(x_ref, wg_ref, w_ref, pp_ref, off_ref, nb_ref, cnt_ref):
    x = x_ref[...]
    logits = jnp.dot(x, wg_ref[...], preferred_element_type=jnp.float32)
    capped = SOFTCAP * jnp.tanh(logits / SOFTCAP)
    probs = jax.nn.softmax(capped, axis=-1)
    i1 = jnp.argmax(probs, axis=-1)
    w1 = jnp.max(probs, axis=-1)
    cols = jax.lax.broadcasted_iota(jnp.int32, probs.shape, 1)
    masked = jnp.where(cols == i1[:, None], -jnp.inf, probs)
    i2 = jnp.argmax(masked, axis=-1)
    w2 = jnp.max(masked, axis=-1)
    w_ref[...] = jnp.stack([w1, w2], axis=-1)

    flat_e = jnp.concatenate([i1[:, None], i2[:, None]], axis=0).astype(jnp.int32)
    erange = jax.lax.broadcasted_iota(jnp.int32, (A, E), 1)
    oh = (flat_e == erange).astype(jnp.float32)
    ic = _shift_cumsum(oh)
    rank = jnp.sum(ic * oh, axis=-1) - 1.0
    counts = ic[A - 1, :]

    c16 = jnp.floor((counts + 15.0) / 16.0) * 16.0
    tri_lo = (jax.lax.broadcasted_iota(jnp.int32, (E, E), 0)
              < jax.lax.broadcasted_iota(jnp.int32, (E, E), 1)).astype(jnp.float32)
    g16 = jnp.dot(c16[None, :], tri_lo,
                  preferred_element_type=jnp.float32)[0]
    pp = jnp.sum(oh * g16[None, :], axis=-1) + rank
    pp_ref[...] = pp.astype(jnp.int32).reshape(TOPK, T)
    off_ref[...] = g16[None, :].astype(jnp.int32)
    nb_ref[...] = jnp.floor((c16[None, :] + 63.0) / 64.0).astype(jnp.int32)
    cnt_ref[...] = counts[None, :].astype(jnp.int32)


def _route(x, wg):
    return pl.pallas_call(
        _route_body,
        out_shape=(
            jax.ShapeDtypeStruct((T, TOPK), jnp.float32),
            jax.ShapeDtypeStruct((TOPK, T), jnp.int32),
            jax.ShapeDtypeStruct((1, E), jnp.int32),
            jax.ShapeDtypeStruct((1, E), jnp.int32),
            jax.ShapeDtypeStruct((1, E), jnp.int32),
        ),
    )(x, wg)


def _ffn_body(pp_ref, off_ref, nb_ref, cnt_ref, x_hbm, w1_ref, w3_ref, w2_ref,
              ys_hbm, x_v, ys_v, xb_v, yb_v, rid_s, sem_in, sem_out):
    e = pl.program_id(0)

    @pl.when(e == 0)
    def _():
        pltpu.make_async_copy(x_hbm, x_v, sem_in).start()

        def fill(a, _):
            rid_s[pp_ref[a]] = a
            return 0

        jax.lax.fori_loop(0, A, fill, 0)
        pltpu.make_async_copy(x_hbm, x_v, sem_in).wait()

    row0 = off_ref[e]

    def step(k, _):
        base = row0 + 64 * k
        nv = jnp.minimum(cnt_ref[e] - 64 * k, 64)

        def gather(r, _):
            a = rid_s[base + r]
            tok = jax.lax.bitwise_and(a, T - 1)
            xb_v[pl.ds(r, 1), :] = x_v[pl.ds(tok, 1), :]
            return 0

        jax.lax.fori_loop(0, nv, gather, 0)

        xb = xb_v[...].astype(jnp.bfloat16)
        h = jax.nn.gelu(
            jnp.dot(xb, w1_ref[0].astype(jnp.bfloat16),
                    preferred_element_type=jnp.float32)
        ) * jnp.dot(xb, w3_ref[0].astype(jnp.bfloat16),
                    preferred_element_type=jnp.float32)
        yb_v[...] = jnp.dot(h.astype(jnp.bfloat16),
                            w2_ref[0].astype(jnp.bfloat16),
                            preferred_element_type=jnp.float32)

        def scatter(r, _):
            a = rid_s[base + r]
            ys_v[pl.ds(a, 1), :] = yb_v[pl.ds(r, 1), :]
            return 0

        jax.lax.fori_loop(0, nv, scatter, 0)
        return 0

    jax.lax.fori_loop(0, nb_ref[e], step, 0)

    @pl.when(e == E - 1)
    def _():
        pltpu.make_async_copy(ys_v, ys_hbm, sem_out).start()
        pltpu.make_async_copy(ys_v, ys_hbm, sem_out).wait()


def _ffn(x, w1, w3, w2, pp_flat, off, nb64, cnt):
    grid_spec = pltpu.PrefetchScalarGridSpec(
        num_scalar_prefetch=4,
        grid=(E,),
        in_specs=[
            pl.BlockSpec(memory_space=pltpu.MemorySpace.HBM),
            pl.BlockSpec((1, D, DFF), lambda e, *_: (e, 0, 0)),
            pl.BlockSpec((1, D, DFF), lambda e, *_: (e, 0, 0)),
            pl.BlockSpec((1, DFF, D), lambda e, *_: (e, 0, 0)),
        ],
        out_specs=pl.BlockSpec(memory_space=pltpu.MemorySpace.HBM),
        scratch_shapes=[
            pltpu.VMEM((T, D), jnp.float32),
            pltpu.VMEM((A, D), jnp.float32),
            pltpu.VMEM((64, D), jnp.float32),
            pltpu.VMEM((64, D), jnp.float32),
            pltpu.SMEM((NP,), jnp.int32),
            pltpu.SemaphoreType.DMA,
            pltpu.SemaphoreType.DMA,
        ],
    )
    return pl.pallas_call(
        _ffn_body,
        grid_spec=grid_spec,
        out_shape=jax.ShapeDtypeStruct((A, D), jnp.float32),
    )(pp_flat, off, nb64, cnt, x, w1, w3, w2)


def kernel(hidden_states, Wg, W1, W3, W2):
    x = hidden_states
    topk_w, pp, off, nb64, cnt = _route(x, Wg)

    ys = _ffn(x, W1, W3, W2, pp.reshape(-1), off[0], nb64[0], cnt[0])
    ys = ys.reshape(TOPK, T, D)

    out = topk_w[:, 0:1] * ys[0] + topk_w[:, 1:2] * ys[1]
    return out


# speedup vs baseline: 1.0270x; 1.0249x over previous
import jax
import jax.numpy as jnp
from jax.experimental import pallas as pl
from jax.experimental.pallas import tpu as pltpu

E = 64
TOPK = 2
D = 1024
DFF = 1024
T = 2048
SOFTCAP = 30.0

A = T * TOPK
NP = 5120


def _shift_cumsum(a):
    n = a.shape[0]
    s = 1
    while s < n:
        a = a + jnp.concatenate([jnp.zeros((s,) + a.shape[1:], a.dtype), a[:-s]], axis=0)
        s *= 2
    return a


def _route_body(x_ref, wg_ref, w_ref, pp_ref, off_ref, nb_ref, cnt_ref):
    x = x_ref[...]
    logits = jnp.dot(x, wg_ref[...], preferred_element_type=jnp.float32)
    capped = SOFTCAP * jnp.tanh(logits / SOFTCAP)
    probs = jax.nn.softmax(capped, axis=-1)
    i1 = jnp.argmax(probs, axis=-1)
    w1 = jnp.max(probs, axis=-1)
    cols = jax.lax.broadcasted_iota(jnp.int32, probs.shape, 1)
    masked = jnp.where(cols == i1[:, None], -jnp.inf, probs)
    i2 = jnp.argmax(masked, axis=-1)
    w2 = jnp.max(masked, axis=-1)
    w_ref[...] = jnp.stack([w1, w2], axis=-1)

    flat_e = jnp.concatenate([i1[:, None], i2[:, None]], axis=0).astype(jnp.int32)
    erange = jax.lax.broadcasted_iota(jnp.int32, (A, E), 1)
    oh = (flat_e == erange).astype(jnp.float32)
    ic = _shift_cumsum(oh)
    rank = jnp.sum(ic * oh, axis=-1) - 1.0
    counts = ic[A - 1, :]

    c16 = jnp.floor((counts + 15.0) / 16.0) * 16.0
    tri_lo = (jax.lax.broadcasted_iota(jnp.int32, (E, E), 0)
              < jax.lax.broadcasted_iota(jnp.int32, (E, E), 1)).astype(jnp.float32)
    g16 = jnp.dot(c16[None, :], tri_lo,
                  preferred_element_type=jnp.float32)[0]
    pp = jnp.sum(oh * g16[None, :], axis=-1) + rank
    pp_ref[...] = pp.astype(jnp.int32).reshape(TOPK, T)
    off_ref[...] = g16[None, :].astype(jnp.int32)
    nb_ref[...] = jnp.floor((c16[None, :] + 63.0) / 64.0).astype(jnp.int32)
    cnt_ref[...] = counts[None, :].astype(jnp.int32)


def _route(x, wg):
    return pl.pallas_call(
        _route_body,
        out_shape=(
            jax.ShapeDtypeStruct((T, TOPK), jnp.float32),
            jax.ShapeDtypeStruct((TOPK, T), jnp.int32),
            jax.ShapeDtypeStruct((1, E), jnp.int32),
            jax.ShapeDtypeStruct((1, E), jnp.int32),
            jax.ShapeDtypeStruct((1, E), jnp.int32),
        ),
    )(x, wg)


def _ffn_body(pp_ref, off_ref, nb_ref, cnt_ref, x_hbm, w1_ref, w3_ref, w2_ref,
              ys_hbm, x_v, ys_v, xb_v, yb_v, rid_s, sem_in, sem_out):
    e = pl.program_id(0)

    @pl.when(e == 0)
    def _():
        pltpu.make_async_copy(x_hbm, x_v, sem_in).start()

        def fill(a, _):
            rid_s[pp_ref[a]] = a
            return 0

        jax.lax.fori_loop(0, A, fill, 0)
        pltpu.make_async_copy(x_hbm, x_v, sem_in).wait()

    row0 = off_ref[e]

    def step(k, _):
        base = row0 + 64 * k
        nv = jnp.minimum(cnt_ref[e] - 64 * k, 64)

        for r in range(64):
            a = rid_s[base + r]
            tok = jax.lax.bitwise_and(a, T - 1)
            xb_v[pl.ds(r, 1), :] = x_v[pl.ds(tok, 1), :]

        xb = xb_v[...]
        h = jax.nn.gelu(
            jnp.dot(xb, w1_ref[0], preferred_element_type=jnp.float32)
        ) * jnp.dot(xb, w3_ref[0], preferred_element_type=jnp.float32)
        yb_v[...] = jnp.dot(h, w2_ref[0], preferred_element_type=jnp.float32)

        for r in range(64):
            a = jax.lax.bitwise_and(rid_s[base + r], A - 1)
            dst = jnp.where(r < nv, a, A + r)
            ys_v[pl.ds(dst, 1), :] = yb_v[pl.ds(r, 1), :]
        return 0

    jax.lax.fori_loop(0, nb_ref[e], step, 0)

    @pl.when(e == E - 1)
    def _():
        pltpu.make_async_copy(ys_v, ys_hbm, sem_out).start()
        pltpu.make_async_copy(ys_v, ys_hbm, sem_out).wait()


def _ffn(x, w1, w3, w2, pp_flat, off, nb64, cnt):
    grid_spec = pltpu.PrefetchScalarGridSpec(
        num_scalar_prefetch=4,
        grid=(E,),
        in_specs=[
            pl.BlockSpec(memory_space=pltpu.MemorySpace.HBM),
            pl.BlockSpec((1, D, DFF), lambda e, *_: (e, 0, 0)),
            pl.BlockSpec((1, D, DFF), lambda e, *_: (e, 0, 0)),
            pl.BlockSpec((1, DFF, D), lambda e, *_: (e, 0, 0)),
        ],
        out_specs=pl.BlockSpec(memory_space=pltpu.MemorySpace.HBM),
        scratch_shapes=[
            pltpu.VMEM((T, D), jnp.float32),
            pltpu.VMEM((A + 64, D), jnp.float32),
            pltpu.VMEM((64, D), jnp.float32),
            pltpu.VMEM((64, D), jnp.float32),
            pltpu.SMEM((NP,), jnp.int32),
            pltpu.SemaphoreType.DMA,
            pltpu.SemaphoreType.DMA,
        ],
    )
    return pl.pallas_call(
        _ffn_body,
        grid_spec=grid_spec,
        out_shape=jax.ShapeDtypeStruct((A + 64, D), jnp.float32),
    )(pp_flat, off, nb64, cnt, x, w1, w3, w2)


def kernel(hidden_states, Wg, W1, W3, W2):
    x = hidden_states
    topk_w, pp, off, nb64, cnt = _route(x, Wg)

    ys = _ffn(x, W1, W3, W2, pp.reshape(-1), off[0], nb64[0], cnt[0])
    ys = ys[:A].reshape(TOPK, T, D)

    out = topk_w[:, 0:1] * ys[0] + topk_w[:, 1:2] * ys[1]
    return out


# speedup vs baseline: 1.0845x; 1.0559x over previous
import jax
import jax.numpy as jnp
from jax.experimental import pallas as pl
from jax.experimental.pallas import tpu as pltpu

E = 64
TOPK = 2
D = 1024
DFF = 1024
T = 2048
SOFTCAP = 30.0

A = T * TOPK
NP = 5120


def _shift_cumsum(a):
    n = a.shape[0]
    s = 1
    while s < n:
        a = a + jnp.concatenate([jnp.zeros((s,) + a.shape[1:], a.dtype), a[:-s]], axis=0)
        s *= 2
    return a


def _route_body(x_ref, wg_ref, w_ref, pp_ref, off_ref, nb_ref, cnt_ref):
    x = x_ref[...]
    logits = jnp.dot(x, wg_ref[...], preferred_element_type=jnp.float32)
    capped = SOFTCAP * jnp.tanh(logits / SOFTCAP)
    probs = jax.nn.softmax(capped, axis=-1)
    i1 = jnp.argmax(probs, axis=-1)
    w1 = jnp.max(probs, axis=-1)
    cols = jax.lax.broadcasted_iota(jnp.int32, probs.shape, 1)
    masked = jnp.where(cols == i1[:, None], -jnp.inf, probs)
    i2 = jnp.argmax(masked, axis=-1)
    w2 = jnp.max(masked, axis=-1)
    w_ref[...] = jnp.stack([w1, w2], axis=-1)

    flat_e = jnp.concatenate([i1[:, None], i2[:, None]], axis=0).astype(jnp.int32)
    erange = jax.lax.broadcasted_iota(jnp.int32, (A, E), 1)
    oh = (flat_e == erange).astype(jnp.float32)
    ic = _shift_cumsum(oh)
    rank = jnp.sum(ic * oh, axis=-1) - 1.0
    counts = ic[A - 1, :]

    c16 = jnp.floor((counts + 15.0) / 16.0) * 16.0
    tri_lo = (jax.lax.broadcasted_iota(jnp.int32, (E, E), 0)
              < jax.lax.broadcasted_iota(jnp.int32, (E, E), 1)).astype(jnp.float32)
    g16 = jnp.dot(c16[None, :], tri_lo,
                  preferred_element_type=jnp.float32)[0]
    pp = jnp.sum(oh * g16[None, :], axis=-1) + rank
    pp_ref[...] = pp.astype(jnp.int32).reshape(TOPK, T)
    off_ref[...] = g16[None, :].astype(jnp.int32)
    nb_ref[...] = jnp.floor((c16[None, :] + 63.0) / 64.0).astype(jnp.int32)
    cnt_ref[...] = counts[None, :].astype(jnp.int32)


def _route(x, wg):
    return pl.pallas_call(
        _route_body,
        out_shape=(
            jax.ShapeDtypeStruct((T, TOPK), jnp.float32),
            jax.ShapeDtypeStruct((TOPK, T), jnp.int32),
            jax.ShapeDtypeStruct((1, E), jnp.int32),
            jax.ShapeDtypeStruct((1, E), jnp.int32),
            jax.ShapeDtypeStruct((1, E), jnp.int32),
        ),
    )(x, wg)


def _ffn_body(pp_ref, off_ref, nb_ref, cnt_ref, x_hbm, tw_ref, w1_ref, w3_ref,
              w2_ref, o_hbm, x_v, ys_v, xb_v, yb_v, rid_s, sem_in, sem_out):
    e = pl.program_id(0)

    @pl.when(e == 0)
    def _():
        pltpu.make_async_copy(x_hbm, x_v, sem_in).start()

        def fill(a, _):
            rid_s[pp_ref[a]] = a
            return 0

        jax.lax.fori_loop(0, A, fill, 0)
        pltpu.make_async_copy(x_hbm, x_v, sem_in).wait()

    row0 = off_ref[e]

    def step(k, _):
        base = row0 + 64 * k
        nv = jnp.minimum(cnt_ref[e] - 64 * k, 64)

        for r in range(64):
            a = rid_s[base + r]
            tok = jax.lax.bitwise_and(a, T - 1)
            xb_v[pl.ds(r, 1), :] = x_v[pl.ds(tok, 1), :]

        xb = xb_v[...]
        h = jax.nn.gelu(
            jnp.dot(xb, w1_ref[0], preferred_element_type=jnp.float32)
        ) * jnp.dot(xb, w3_ref[0], preferred_element_type=jnp.float32)
        yb_v[...] = jnp.dot(h, w2_ref[0], preferred_element_type=jnp.float32)

        for r in range(64):
            a = jax.lax.bitwise_and(rid_s[base + r], A - 1)
            dst = jnp.where(r < nv, a, A + r)
            ys_v[pl.ds(dst, 1), :] = yb_v[pl.ds(r, 1), :]
        return 0

    jax.lax.fori_loop(0, nb_ref[e], step, 0)

    @pl.when(e == E - 1)
    def _():
        tw = tw_ref[...]
        x_v[...] = (tw[:, 0:1] * ys_v[0:T, :] + tw[:, 1:2] * ys_v[T:2 * T, :])
        pltpu.make_async_copy(x_v, o_hbm, sem_out).start()
        pltpu.make_async_copy(x_v, o_hbm, sem_out).wait()


def _ffn(x, topk_w, w1, w3, w2, pp_flat, off, nb64, cnt):
    grid_spec = pltpu.PrefetchScalarGridSpec(
        num_scalar_prefetch=4,
        grid=(E,),
        in_specs=[
            pl.BlockSpec(memory_space=pltpu.MemorySpace.HBM),
            pl.BlockSpec((T, TOPK), lambda e, *_: (0, 0)),
            pl.BlockSpec((1, D, DFF), lambda e, *_: (e, 0, 0)),
            pl.BlockSpec((1, D, DFF), lambda e, *_: (e, 0, 0)),
            pl.BlockSpec((1, DFF, D), lambda e, *_: (e, 0, 0)),
        ],
        out_specs=pl.BlockSpec(memory_space=pltpu.MemorySpace.HBM),
        scratch_shapes=[
            pltpu.VMEM((T, D), jnp.float32),
            pltpu.VMEM((A + 64, D), jnp.float32),
            pltpu.VMEM((64, D), jnp.float32),
            pltpu.VMEM((64, D), jnp.float32),
            pltpu.SMEM((NP,), jnp.int32),
            pltpu.SemaphoreType.DMA,
            pltpu.SemaphoreType.DMA,
        ],
    )
    return pl.pallas_call(
        _ffn_body,
        grid_spec=grid_spec,
        out_shape=jax.ShapeDtypeStruct((T, D), jnp.float32),
    )(pp_flat, off, nb64, cnt, x, topk_w, w1, w3, w2)


def kernel(hidden_states, Wg, W1, W3, W2):
    x = hidden_states
    topk_w, pp, off, nb64, cnt = _route(x, Wg)
    return _ffn(x, topk_w, W1, W3, W2, pp.reshape(-1), off[0], nb64[0], cnt[0])
